# baseline (speedup 1.0000x reference)
import functools

import jax
import jax.numpy as jnp
from jax.experimental import pallas as pl
from jax.experimental.pallas import tpu as pltpu


def _dt(a, b):
    return jnp.dot(a.astype(b.dtype), b, preferred_element_type=jnp.float32)


def _lrelu(x):
    return jnp.where(x > 0, x, 0.01 * x)


def _elu(x):
    return jnp.where(x > 0, x, jnp.exp(jnp.minimum(x, 0.0)) - 1.0)


def _gru_step(WihT, WhhT, bih, bhh, x, h):
    x16 = x.astype(WihT.dtype)
    h16 = h.astype(WihT.dtype)
    r = jax.nn.sigmoid(_dt(x16, WihT[0]) + bih[0:1] + _dt(h16, WhhT[0]) + bhh[0:1])
    z = jax.nn.sigmoid(_dt(x16, WihT[1]) + bih[1:2] + _dt(h16, WhhT[1]) + bhh[1:2])
    n = jnp.tanh(_dt(x16, WihT[2]) + bih[2:3] + r * (_dt(h16, WhhT[2]) + bhh[2:3]))
    return (1.0 - z) * n + z * h


def _fp_kernel(radius, nfp,
               atom_ref, bond_ref, adegc_ref, bdegc_ref, am_ref,
               seg_ref, segt_ref,
               atomWT_ref, atomb_ref, nfcaWT_ref, nfcbWT_ref, nfcb_ref,
               alignWa_ref, alignWn_ref, alignb_ref,
               attendWT_ref, attendb_ref,
               gruWihT_ref, gruWhhT_ref, grubih_ref, grubhh_ref,
               mgWihT_ref, mgWhhT_ref, mgbih_ref, mgbhh_ref,
               molWm_ref, molWa_ref, molb_ref, mattWT_ref, mattb_ref,
               mol_out_ref, maw_out_ref, act_out_ref, af_out_ref):
    atom = atom_ref[0]
    bond = bond_ref[0]
    adegc = adegc_ref[0]
    bdegc = bdegc_ref[0]
    am = am_ref[0]
    SEG = seg_ref[...]
    SEGT = segt_ref[...]

    L, AF = atom.shape
    NB = bond.shape[0]
    LM = adegc.shape[0]
    D = atomWT_ref.shape[-1]

    oh_atom = (jax.lax.broadcasted_iota(jnp.int32, (LM, L), 1)
               == adegc).astype(jnp.bfloat16)
    oh_bond = (jax.lax.broadcasted_iota(jnp.int32, (LM, NB), 1)
               == bdegc).astype(jnp.bfloat16)
    amask = 1.0 - oh_atom[:, L - 1:L].astype(jnp.float32)
    smask = -9e8 * oh_atom[:, L - 1:L].astype(jnp.float32)
    an = _dt(oh_atom, atom.astype(jnp.bfloat16))
    bn = _dt(oh_bond, bond.astype(jnp.bfloat16))
    mmask = jnp.where(am == 0.0, -9e8, 0.0)

    for f in range(nfp):
        atom_feature = _lrelu(_dt(atom, atomWT_ref[f]) + atomb_ref[f])
        nbr = _lrelu(_dt(an, nfcaWT_ref[f]) + _dt(bn, nfcbWT_ref[f])
                     + nfcb_ref[f])
        h = atom_feature
        act = h
        for r in range(radius):
            wa = alignWa_ref[f, r:r + 1, :]
            wn = alignWn_ref[f, r:r + 1, :]
            b = alignb_ref[f, r]
            sA = jnp.sum(h * wa, axis=1, keepdims=True)
            sAc = _dt(SEGT, sA.astype(jnp.bfloat16))
            sNc = jnp.sum(nbr * wn, axis=1, keepdims=True)
            score = _lrelu(sAc + sNc + b) + smask
            em = jnp.exp(score - jnp.max(score)) * amask
            gsum = _dt(SEG, em.astype(jnp.bfloat16))
            nft = _dt(nbr, attendWT_ref[f, r]) + attendb_ref[f, r:r + 1, :]
            ctx_raw = _dt(SEG, (nft * em).astype(jnp.bfloat16))
            ctx = jnp.where(gsum > 0, ctx_raw / jnp.maximum(gsum, 1e-38), 0.0)
            context = _elu(ctx)
            h = _gru_step(gruWihT_ref[f, r], gruWhhT_ref[f, r],
                          grubih_ref[f, r], grubhh_ref[f, r], context, h)
            act = jax.nn.relu(h)
            if r < radius - 1:
                nbr = _dt(oh_atom, act.astype(jnp.bfloat16))

        mol_feature = jnp.sum(act * am, axis=0, keepdims=True)
        act_mol = jax.nn.relu(mol_feature)
        sA2 = jnp.sum(act_mol * molWm_ref[f], axis=1, keepdims=True)
        sN2 = jnp.sum(act * molWa_ref[f], axis=1, keepdims=True)
        mas = _lrelu(sN2 + sA2 + molb_ref[f, 0]) + mmask
        mmax = jnp.max(mas, axis=0, keepdims=True)
        me = jnp.exp(mas - mmax)
        maw = me / jnp.sum(me, axis=0, keepdims=True) * am
        aft = _dt(act, mattWT_ref[f]) + mattb_ref[f]
        mol_context = _elu(jnp.sum(maw * aft, axis=0, keepdims=True))
        mol_feature = _gru_step(mgWihT_ref[f], mgWhhT_ref[f],
                                mgbih_ref[f], mgbhh_ref[f], mol_context, mol_feature)

        mol_out_ref[f, 0] = mol_feature
        maw_out_ref[f, 0] = maw
        act_out_ref[f, 0] = act
        af_out_ref[f, 0] = h


def _comb_kernel(mol_ref, gWT_ref, gb_ref, f1WT_ref, f1b_ref, f2w_ref, f2b_ref,
                 sel1_ref, sel2_ref, o1_ref, o2_ref):
    mol = mol_ref[...]
    so = mol[0]
    experts = (mol[1], mol[2])
    gates = (mol[3], mol[4])
    outs = ((sel1_ref, o1_ref), (sel2_ref, o2_ref))
    for g in range(2):
        logits = jnp.dot(gates[g], gWT_ref[g]) + gb_ref[g]
        lmax = jnp.max(logits, axis=1, keepdims=True)
        e = jnp.exp(logits - lmax)
        sel = e / jnp.sum(e, axis=1, keepdims=True)
        sel_ref, o_ref = outs[g]
        sel_ref[...] = sel
        gate_out = experts[g] * sel[:, 0:1] + so * sel[:, 1:2]
        hdn = jax.nn.relu(jnp.dot(gate_out, f1WT_ref[g]) + f1b_ref[g])
        o_ref[...] = jnp.sum(hdn * f2w_ref[g], axis=1, keepdims=True) + f2b_ref[g]


def kernel(atom_list, bond_list, atom_degree_list, bond_degree_list, atom_mask, params):
    B, L, AF = atom_list.shape
    NB, BF = bond_list.shape[1], bond_list.shape[2]
    M = atom_degree_list.shape[-1]
    LM = L * M
    D = params["shared"]["atom_fc"]["W"].shape[0]
    fps = [params["shared"], params["task1"], params["task2"],
           params["gate1"]["fp"], params["gate2"]["fp"]]
    NFP = len(fps)
    R = len(fps[0]["gru"])

    atomWT = jnp.stack([p["atom_fc"]["W"].T for p in fps])
    atomb = jnp.stack([p["atom_fc"]["b"] for p in fps])[:, None, :]
    nWT = jnp.stack([p["neighbor_fc"]["W"].T for p in fps])
    nfcaWT = nWT[:, :AF, :]
    nfcbWT = nWT[:, AF:, :]
    nfcb = jnp.stack([p["neighbor_fc"]["b"] for p in fps])[:, None, :]
    alignW = jnp.stack([jnp.stack([p["align"][r]["W"][0] for r in range(R)])
                        for p in fps])
    alignWa = alignW[:, :, :D]
    alignWn = alignW[:, :, D:]
    alignb = jnp.stack([jnp.stack([p["align"][r]["b"][0] for r in range(R)])
                        for p in fps])
    attendWT = jnp.stack([jnp.stack([p["attend"][r]["W"].T for r in range(R)])
                          for p in fps])
    attendb = jnp.stack([jnp.stack([p["attend"][r]["b"] for r in range(R)])
                         for p in fps])
    gruWihT = jnp.stack([jnp.stack([p["gru"][r]["Wih"].reshape(3, D, D).transpose(0, 2, 1)
                                    for r in range(R)]) for p in fps])
    gruWhhT = jnp.stack([jnp.stack([p["gru"][r]["Whh"].reshape(3, D, D).transpose(0, 2, 1)
                                    for r in range(R)]) for p in fps])
    grubih = jnp.stack([jnp.stack([p["gru"][r]["bih"].reshape(3, D)
                                   for r in range(R)]) for p in fps])
    grubhh = jnp.stack([jnp.stack([p["gru"][r]["bhh"].reshape(3, D)
                                   for r in range(R)]) for p in fps])
    mgWihT = jnp.stack([p["mol_gru"]["Wih"].reshape(3, D, D).transpose(0, 2, 1)
                        for p in fps])
    mgWhhT = jnp.stack([p["mol_gru"]["Whh"].reshape(3, D, D).transpose(0, 2, 1)
                        for p in fps])
    mgbih = jnp.stack([p["mol_gru"]["bih"].reshape(3, D) for p in fps])
    mgbhh = jnp.stack([p["mol_gru"]["bhh"].reshape(3, D) for p in fps])
    molalignW = jnp.stack([p["mol_align"]["W"][0] for p in fps])
    molWm = molalignW[:, None, :D]
    molWa = molalignW[:, None, D:]
    molb = jnp.stack([p["mol_align"]["b"][0] for p in fps])[:, None]
    mattWT = jnp.stack([p["mol_attend"]["W"].T for p in fps])
    atomWT, nfcaWT, nfcbWT, attendWT, gruWihT, gruWhhT = (
        a.astype(jnp.bfloat16) for a in
        (atomWT, nfcaWT, nfcbWT, attendWT, gruWihT, gruWhhT))
    mattb = jnp.stack([p["mol_attend"]["b"] for p in fps])[:, None, :]

    am_col = atom_mask[..., None]
    adegc = atom_degree_list.astype(jnp.int32).reshape(B, LM, 1)
    bdegc = bond_degree_list.astype(jnp.int32).reshape(B, LM, 1)
    lidx = jnp.arange(LM, dtype=jnp.int32) // M
    seg = (lidx[None, :] == jnp.arange(L, dtype=jnp.int32)[:, None]).astype(jnp.bfloat16)
    segt = seg.T

    def dmap(ndim):
        zeros = (0,) * (ndim - 1)
        return lambda b: (b,) + zeros

    def cmap(ndim):
        zeros = (0,) * ndim
        return lambda b: zeros

    vspec = lambda shape, im: pl.BlockSpec(shape, im)
    in_specs = [
        vspec((1, L, AF), dmap(3)),
        vspec((1, NB, BF), dmap(3)),
        vspec((1, LM, 1), dmap(3)),
        vspec((1, LM, 1), dmap(3)),
        vspec((1, L, 1), dmap(3)),
        vspec((L, LM), cmap(2)),
        vspec((LM, L), cmap(2)),
        vspec((NFP, AF, D), cmap(3)),
        vspec((NFP, 1, D), cmap(3)),
        vspec((NFP, AF, D), cmap(3)),
        vspec((NFP, BF, D), cmap(3)),
        vspec((NFP, 1, D), cmap(3)),
        vspec((NFP, R, D), cmap(3)),
        vspec((NFP, R, D), cmap(3)),
        pl.BlockSpec(memory_space=pltpu.SMEM),
        vspec((NFP, R, D, D), cmap(4)),
        vspec((NFP, R, D), cmap(3)),
        vspec((NFP, R, 3, D, D), cmap(5)),
        vspec((NFP, R, 3, D, D), cmap(5)),
        vspec((NFP, R, 3, D), cmap(4)),
        vspec((NFP, R, 3, D), cmap(4)),
        vspec((NFP, 3, D, D), cmap(4)),
        vspec((NFP, 3, D, D), cmap(4)),
        vspec((NFP, 3, D), cmap(3)),
        vspec((NFP, 3, D), cmap(3)),
        vspec((NFP, 1, D), cmap(3)),
        vspec((NFP, 1, D), cmap(3)),
        pl.BlockSpec(memory_space=pltpu.SMEM),
        vspec((NFP, D, D), cmap(3)),
        vspec((NFP, 1, D), cmap(3)),
    ]
    out_specs = [
        vspec((NFP, 1, 1, D), lambda b: (0, b, 0, 0)),
        vspec((NFP, 1, L, 1), lambda b: (0, b, 0, 0)),
        vspec((NFP, 1, L, D), lambda b: (0, b, 0, 0)),
        vspec((NFP, 1, L, D), lambda b: (0, b, 0, 0)),
    ]
    out_shape = [
        jax.ShapeDtypeStruct((NFP, B, 1, D), jnp.float32),
        jax.ShapeDtypeStruct((NFP, B, L, 1), jnp.float32),
        jax.ShapeDtypeStruct((NFP, B, L, D), jnp.float32),
        jax.ShapeDtypeStruct((NFP, B, L, D), jnp.float32),
    ]

    mol_all, maw_all, act_all, af_all = pl.pallas_call(
        functools.partial(_fp_kernel, R, NFP),
        grid=(B,),
        in_specs=in_specs,
        out_specs=out_specs,
        out_shape=out_shape,
    )(atom_list, bond_list, adegc, bdegc, am_col, seg, segt,
      atomWT, atomb, nfcaWT, nfcbWT, nfcb,
      alignWa, alignWn, alignb,
      attendWT, attendb,
      gruWihT, gruWhhT, grubih, grubhh,
      mgWihT, mgWhhT, mgbih, mgbhh,
      molWm, molWa, molb, mattWT, mattb)

    gWT = jnp.stack([params["gate1"]["dnn"]["W"].T, params["gate2"]["dnn"]["W"].T])
    gb = jnp.stack([params["gate1"]["dnn"]["b"], params["gate2"]["dnn"]["b"]])[:, None, :]
    f1WT = jnp.stack([params["tower1"]["fc1"]["W"].T, params["tower2"]["fc1"]["W"].T])
    f1b = jnp.stack([params["tower1"]["fc1"]["b"], params["tower2"]["fc1"]["b"]])[:, None, :]
    f2w = jnp.stack([params["tower1"]["fc2"]["W"][0], params["tower2"]["fc2"]["W"][0]])[:, None, :]
    f2b = jnp.stack([params["tower1"]["fc2"]["b"], params["tower2"]["fc2"]["b"]])[:, None, :]

    mol5 = mol_all.reshape(NFP, B, D)
    full = lambda shape: pl.BlockSpec(shape, lambda: (0,) * len(shape))
    sel1, sel2, o1, o2 = pl.pallas_call(
        _comb_kernel,
        grid=(),
        in_specs=[full(s) for s in ((NFP, B, D), (2, D, 2), (2, 1, 2),
                                    (2, D, 32), (2, 1, 32), (2, 1, 32), (2, 1, 1))],
        out_specs=[full(s) for s in ((B, 2), (B, 2), (B, 1), (B, 1))],
        out_shape=[
            jax.ShapeDtypeStruct((B, 2), jnp.float32),
            jax.ShapeDtypeStruct((B, 2), jnp.float32),
            jax.ShapeDtypeStruct((B, 1), jnp.float32),
            jax.ShapeDtypeStruct((B, 1), jnp.float32),
        ],
    )(mol5, gWT, gb, f1WT, f1b, f2w, f2b)

    out = jnp.concatenate([o1, o2], axis=1)
    att = [maw_all[0], maw_all[1], maw_all[2], sel1, sel2]
    fea_relu = [act_all[0], act_all[1], act_all[2]]
    fea = [af_all[0], af_all[1], af_all[2]]
    return out, att, fea_relu, fea


# speedup vs baseline: 1.0037x; 1.0037x over previous
import functools

import jax
import jax.numpy as jnp
from jax.experimental import pallas as pl
from jax.experimental.pallas import tpu as pltpu


def _lrelu(x):
    return jnp.where(x > 0, x, 0.01 * x)


def _elu(x):
    return jnp.where(x > 0, x, jnp.exp(jnp.minimum(x, 0.0)) - 1.0)


def _gru_step(WihT, WhhT, bih, bhh, x, h):
    r = jax.nn.sigmoid(jnp.dot(x, WihT[0]) + bih[0:1] + jnp.dot(h, WhhT[0]) + bhh[0:1])
    z = jax.nn.sigmoid(jnp.dot(x, WihT[1]) + bih[1:2] + jnp.dot(h, WhhT[1]) + bhh[1:2])
    n = jnp.tanh(jnp.dot(x, WihT[2]) + bih[2:3] + r * (jnp.dot(h, WhhT[2]) + bhh[2:3]))
    return (1.0 - z) * n + z * h


def _fp_kernel(radius, nfp,
               atom_ref, bond_ref, adegc_ref, bdegc_ref, am_ref,
               seg_ref, segt_ref,
               atomWT_ref, atomb_ref, nfcaWT_ref, nfcbWT_ref, nfcb_ref,
               alignWa_ref, alignWn_ref, alignb_ref,
               attendWT_ref, attendb_ref,
               gruWihT_ref, gruWhhT_ref, grubih_ref, grubhh_ref,
               mgWihT_ref, mgWhhT_ref, mgbih_ref, mgbhh_ref,
               molWm_ref, molWa_ref, molb_ref, mattWT_ref, mattb_ref,
               mol_out_ref, maw_out_ref, act_out_ref, af_out_ref):
    atom = atom_ref[0]
    bond = bond_ref[0]
    adegc = adegc_ref[0]
    bdegc = bdegc_ref[0]
    am = am_ref[0]
    SEG = seg_ref[...]
    SEGT = segt_ref[...]

    L, AF = atom.shape
    NB = bond.shape[0]
    LM = adegc.shape[0]
    D = atomWT_ref.shape[-1]

    oh_atom = (jax.lax.broadcasted_iota(jnp.int32, (LM, L), 1)
               == adegc).astype(jnp.float32)
    oh_bond = (jax.lax.broadcasted_iota(jnp.int32, (LM, NB), 1)
               == bdegc).astype(jnp.float32)
    amask = 1.0 - oh_atom[:, L - 1:L]
    smask = -9e8 * oh_atom[:, L - 1:L]
    an = jnp.dot(oh_atom, atom)
    bn = jnp.dot(oh_bond, bond)
    mmask = jnp.where(am == 0.0, -9e8, 0.0)

    for f in range(nfp):
        atom_feature = _lrelu(jnp.dot(atom, atomWT_ref[f]) + atomb_ref[f])
        nbr = _lrelu(jnp.dot(an, nfcaWT_ref[f]) + jnp.dot(bn, nfcbWT_ref[f])
                     + nfcb_ref[f])
        h = atom_feature
        act = h
        for r in range(radius):
            wa = alignWa_ref[f, r:r + 1, :]
            wn = alignWn_ref[f, r:r + 1, :]
            b = alignb_ref[f, r]
            sA = jnp.sum(h * wa, axis=1, keepdims=True)
            sAc = jnp.dot(SEGT, sA)
            sNc = jnp.sum(nbr * wn, axis=1, keepdims=True)
            score = _lrelu(sAc + sNc + b) + smask
            em = jnp.exp(score - jnp.max(score)) * amask
            gsum = jnp.dot(SEG, em)
            nft = jnp.dot(nbr, attendWT_ref[f, r]) + attendb_ref[f, r:r + 1, :]
            ctx_raw = jnp.dot(SEG, nft * em)
            ctx = jnp.where(gsum > 0, ctx_raw / jnp.maximum(gsum, 1e-38), 0.0)
            context = _elu(ctx)
            h = _gru_step(gruWihT_ref[f, r], gruWhhT_ref[f, r],
                          grubih_ref[f, r], grubhh_ref[f, r], context, h)
            act = jax.nn.relu(h)
            if r < radius - 1:
                nbr = jnp.dot(oh_atom, act)

        mol_feature = jnp.sum(act * am, axis=0, keepdims=True)
        act_mol = jax.nn.relu(mol_feature)
        sA2 = jnp.sum(act_mol * molWm_ref[f], axis=1, keepdims=True)
        sN2 = jnp.sum(act * molWa_ref[f], axis=1, keepdims=True)
        mas = _lrelu(sN2 + sA2 + molb_ref[f, 0]) + mmask
        mmax = jnp.max(mas, axis=0, keepdims=True)
        me = jnp.exp(mas - mmax)
        maw = me / jnp.sum(me, axis=0, keepdims=True) * am
        aft = jnp.dot(act, mattWT_ref[f]) + mattb_ref[f]
        mol_context = _elu(jnp.sum(maw * aft, axis=0, keepdims=True))
        mol_feature = _gru_step(mgWihT_ref[f], mgWhhT_ref[f],
                                mgbih_ref[f], mgbhh_ref[f], mol_context, mol_feature)

        mol_out_ref[f, 0] = mol_feature
        maw_out_ref[f, 0] = maw
        act_out_ref[f, 0] = act
        af_out_ref[f, 0] = h


def _comb_kernel(mol_ref, gWT_ref, gb_ref, f1WT_ref, f1b_ref, f2w_ref, f2b_ref,
                 sel1_ref, sel2_ref, o1_ref, o2_ref):
    mol = mol_ref[...]
    so = mol[0]
    experts = (mol[1], mol[2])
    gates = (mol[3], mol[4])
    outs = ((sel1_ref, o1_ref), (sel2_ref, o2_ref))
    for g in range(2):
        logits = jnp.dot(gates[g], gWT_ref[g]) + gb_ref[g]
        lmax = jnp.max(logits, axis=1, keepdims=True)
        e = jnp.exp(logits - lmax)
        sel = e / jnp.sum(e, axis=1, keepdims=True)
        sel_ref, o_ref = outs[g]
        sel_ref[...] = sel
        gate_out = experts[g] * sel[:, 0:1] + so * sel[:, 1:2]
        hdn = jax.nn.relu(jnp.dot(gate_out, f1WT_ref[g]) + f1b_ref[g])
        o_ref[...] = jnp.sum(hdn * f2w_ref[g], axis=1, keepdims=True) + f2b_ref[g]


def kernel(atom_list, bond_list, atom_degree_list, bond_degree_list, atom_mask, params):
    B, L, AF = atom_list.shape
    NB, BF = bond_list.shape[1], bond_list.shape[2]
    M = atom_degree_list.shape[-1]
    LM = L * M
    D = params["shared"]["atom_fc"]["W"].shape[0]
    fps = [params["shared"], params["task1"], params["task2"],
           params["gate1"]["fp"], params["gate2"]["fp"]]
    NFP = len(fps)
    R = len(fps[0]["gru"])

    atomWT = jnp.stack([p["atom_fc"]["W"].T for p in fps])
    atomb = jnp.stack([p["atom_fc"]["b"] for p in fps])[:, None, :]
    nWT = jnp.stack([p["neighbor_fc"]["W"].T for p in fps])
    nfcaWT = nWT[:, :AF, :]
    nfcbWT = nWT[:, AF:, :]
    nfcb = jnp.stack([p["neighbor_fc"]["b"] for p in fps])[:, None, :]
    alignW = jnp.stack([jnp.stack([p["align"][r]["W"][0] for r in range(R)])
                        for p in fps])
    alignWa = alignW[:, :, :D]
    alignWn = alignW[:, :, D:]
    alignb = jnp.stack([jnp.stack([p["align"][r]["b"][0] for r in range(R)])
                        for p in fps])
    attendWT = jnp.stack([jnp.stack([p["attend"][r]["W"].T for r in range(R)])
                          for p in fps])
    attendb = jnp.stack([jnp.stack([p["attend"][r]["b"] for r in range(R)])
                         for p in fps])
    gruWihT = jnp.stack([jnp.stack([p["gru"][r]["Wih"].reshape(3, D, D).transpose(0, 2, 1)
                                    for r in range(R)]) for p in fps])
    gruWhhT = jnp.stack([jnp.stack([p["gru"][r]["Whh"].reshape(3, D, D).transpose(0, 2, 1)
                                    for r in range(R)]) for p in fps])
    grubih = jnp.stack([jnp.stack([p["gru"][r]["bih"].reshape(3, D)
                                   for r in range(R)]) for p in fps])
    grubhh = jnp.stack([jnp.stack([p["gru"][r]["bhh"].reshape(3, D)
                                   for r in range(R)]) for p in fps])
    mgWihT = jnp.stack([p["mol_gru"]["Wih"].reshape(3, D, D).transpose(0, 2, 1)
                        for p in fps])
    mgWhhT = jnp.stack([p["mol_gru"]["Whh"].reshape(3, D, D).transpose(0, 2, 1)
                        for p in fps])
    mgbih = jnp.stack([p["mol_gru"]["bih"].reshape(3, D) for p in fps])
    mgbhh = jnp.stack([p["mol_gru"]["bhh"].reshape(3, D) for p in fps])
    molalignW = jnp.stack([p["mol_align"]["W"][0] for p in fps])
    molWm = molalignW[:, None, :D]
    molWa = molalignW[:, None, D:]
    molb = jnp.stack([p["mol_align"]["b"][0] for p in fps])[:, None]
    mattWT = jnp.stack([p["mol_attend"]["W"].T for p in fps])
    mattb = jnp.stack([p["mol_attend"]["b"] for p in fps])[:, None, :]

    am_col = atom_mask[..., None]
    adegc = atom_degree_list.astype(jnp.int32).reshape(B, LM, 1)
    bdegc = bond_degree_list.astype(jnp.int32).reshape(B, LM, 1)
    lidx = jnp.arange(LM, dtype=jnp.int32) // M
    seg = (lidx[None, :] == jnp.arange(L, dtype=jnp.int32)[:, None]).astype(jnp.float32)
    segt = seg.T

    def dmap(ndim):
        zeros = (0,) * (ndim - 1)
        return lambda b: (b,) + zeros

    def cmap(ndim):
        zeros = (0,) * ndim
        return lambda b: zeros

    vspec = lambda shape, im: pl.BlockSpec(shape, im)
    in_specs = [
        vspec((1, L, AF), dmap(3)),
        vspec((1, NB, BF), dmap(3)),
        vspec((1, LM, 1), dmap(3)),
        vspec((1, LM, 1), dmap(3)),
        vspec((1, L, 1), dmap(3)),
        vspec((L, LM), cmap(2)),
        vspec((LM, L), cmap(2)),
        vspec((NFP, AF, D), cmap(3)),
        vspec((NFP, 1, D), cmap(3)),
        vspec((NFP, AF, D), cmap(3)),
        vspec((NFP, BF, D), cmap(3)),
        vspec((NFP, 1, D), cmap(3)),
        vspec((NFP, R, D), cmap(3)),
        vspec((NFP, R, D), cmap(3)),
        pl.BlockSpec(memory_space=pltpu.SMEM),
        vspec((NFP, R, D, D), cmap(4)),
        vspec((NFP, R, D), cmap(3)),
        vspec((NFP, R, 3, D, D), cmap(5)),
        vspec((NFP, R, 3, D, D), cmap(5)),
        vspec((NFP, R, 3, D), cmap(4)),
        vspec((NFP, R, 3, D), cmap(4)),
        vspec((NFP, 3, D, D), cmap(4)),
        vspec((NFP, 3, D, D), cmap(4)),
        vspec((NFP, 3, D), cmap(3)),
        vspec((NFP, 3, D), cmap(3)),
        vspec((NFP, 1, D), cmap(3)),
        vspec((NFP, 1, D), cmap(3)),
        pl.BlockSpec(memory_space=pltpu.SMEM),
        vspec((NFP, D, D), cmap(3)),
        vspec((NFP, 1, D), cmap(3)),
    ]
    out_specs = [
        vspec((NFP, 1, 1, D), lambda b: (0, b, 0, 0)),
        vspec((NFP, 1, L, 1), lambda b: (0, b, 0, 0)),
        vspec((NFP, 1, L, D), lambda b: (0, b, 0, 0)),
        vspec((NFP, 1, L, D), lambda b: (0, b, 0, 0)),
    ]
    out_shape = [
        jax.ShapeDtypeStruct((NFP, B, 1, D), jnp.float32),
        jax.ShapeDtypeStruct((NFP, B, L, 1), jnp.float32),
        jax.ShapeDtypeStruct((NFP, B, L, D), jnp.float32),
        jax.ShapeDtypeStruct((NFP, B, L, D), jnp.float32),
    ]

    mol_all, maw_all, act_all, af_all = pl.pallas_call(
        functools.partial(_fp_kernel, R, NFP),
        grid=(B,),
        in_specs=in_specs,
        out_specs=out_specs,
        out_shape=out_shape,
    )(atom_list, bond_list, adegc, bdegc, am_col, seg, segt,
      atomWT, atomb, nfcaWT, nfcbWT, nfcb,
      alignWa, alignWn, alignb,
      attendWT, attendb,
      gruWihT, gruWhhT, grubih, grubhh,
      mgWihT, mgWhhT, mgbih, mgbhh,
      molWm, molWa, molb, mattWT, mattb)

    gWT = jnp.stack([params["gate1"]["dnn"]["W"].T, params["gate2"]["dnn"]["W"].T])
    gb = jnp.stack([params["gate1"]["dnn"]["b"], params["gate2"]["dnn"]["b"]])[:, None, :]
    f1WT = jnp.stack([params["tower1"]["fc1"]["W"].T, params["tower2"]["fc1"]["W"].T])
    f1b = jnp.stack([params["tower1"]["fc1"]["b"], params["tower2"]["fc1"]["b"]])[:, None, :]
    f2w = jnp.stack([params["tower1"]["fc2"]["W"][0], params["tower2"]["fc2"]["W"][0]])[:, None, :]
    f2b = jnp.stack([params["tower1"]["fc2"]["b"], params["tower2"]["fc2"]["b"]])[:, None, :]

    mol5 = mol_all.reshape(NFP, B, D)
    full = lambda shape: pl.BlockSpec(shape, lambda: (0,) * len(shape))
    sel1, sel2, o1, o2 = pl.pallas_call(
        _comb_kernel,
        grid=(),
        in_specs=[full(s) for s in ((NFP, B, D), (2, D, 2), (2, 1, 2),
                                    (2, D, 32), (2, 1, 32), (2, 1, 32), (2, 1, 1))],
        out_specs=[full(s) for s in ((B, 2), (B, 2), (B, 1), (B, 1))],
        out_shape=[
            jax.ShapeDtypeStruct((B, 2), jnp.float32),
            jax.ShapeDtypeStruct((B, 2), jnp.float32),
            jax.ShapeDtypeStruct((B, 1), jnp.float32),
            jax.ShapeDtypeStruct((B, 1), jnp.float32),
        ],
    )(mol5, gWT, gb, f1WT, f1b, f2w, f2b)

    out = jnp.concatenate([o1, o2], axis=1)
    att = [maw_all[0], maw_all[1], maw_all[2], sel1, sel2]
    fea_relu = [act_all[0], act_all[1], act_all[2]]
    fea = [af_all[0], af_all[1], af_all[2]]
    return out, att, fea_relu, fea
